# baseline (speedup 1.0000x reference)
import functools

import jax
import jax.numpy as jnp
from jax import lax
from jax.experimental import pallas as pl
from jax.experimental.pallas import tpu as pltpu

_H = 128
_NG = 16
_ROW = 2000
_EROW = 2048


def _silu(x):
    return x / (1.0 + jnp.exp(-x))


def _emb_body(x_ref, w_ref, b_ref, o_ref):
    o_ref[...] = jnp.dot(x_ref[...], w_ref[...],
                         preferred_element_type=jnp.float32) + b_ref[...]


@functools.lru_cache(maxsize=None)
def _emb_call(n):
    grid = n // _ROW
    return pl.pallas_call(
        _emb_body,
        grid=(grid,),
        in_specs=[
            pl.BlockSpec((_ROW, _H), lambda i: (i, 0)),
            pl.BlockSpec((_H, _H), lambda i: (0, 0)),
            pl.BlockSpec((1, _H), lambda i: (0, 0)),
        ],
        out_specs=pl.BlockSpec((_ROW, _H), lambda i: (i, 0)),
        out_shape=jax.ShapeDtypeStruct((n, _H), jnp.float32),
    )


def _proj_body(nproj, x_ref, w_ref, *o_refs):
    y = jnp.dot(x_ref[...], w_ref[...], preferred_element_type=jnp.float32)
    for p in range(nproj):
        o_refs[p][0] = y[:, p * _H:p * _H + 64]
        o_refs[p][1] = y[:, p * _H + 64:(p + 1) * _H]


@functools.lru_cache(maxsize=None)
def _proj_call(n, nproj):
    grid = n // _ROW
    return pl.pallas_call(
        functools.partial(_proj_body, nproj),
        grid=(grid,),
        in_specs=[
            pl.BlockSpec((_ROW, _H), lambda i: (i, 0)),
            pl.BlockSpec((_H, nproj * _H), lambda i: (0, 0)),
        ],
        out_specs=[pl.BlockSpec((2, _ROW, 64), lambda i: (0, i, 0))
                   for _ in range(nproj)],
        out_shape=[jax.ShapeDtypeStruct((2, n, 64), jnp.float32)
                   for _ in range(nproj)],
    )


def _cprep_body(inv_ref, w_ref, b_ref, o_ref):
    y = jnp.dot(inv_ref[...], w_ref[...],
                preferred_element_type=jnp.float32) + b_ref[...]
    o_ref[0] = y[:, :64]
    o_ref[1] = y[:, 64:]


@functools.lru_cache(maxsize=None)
def _cprep_call(e):
    grid = e // _EROW
    return pl.pallas_call(
        _cprep_body,
        grid=(grid,),
        in_specs=[
            pl.BlockSpec((_EROW, 3), lambda i: (i, 0)),
            pl.BlockSpec((3, _H), lambda i: (0, 0)),
            pl.BlockSpec((1, _H), lambda i: (0, 0)),
        ],
        out_specs=pl.BlockSpec((2, _EROW, 64), lambda i: (0, i, 0)),
        out_shape=jax.ShapeDtypeStruct((2, e, 64), jnp.float32),
    )


def _upd_body(has_inc, x_ref, ha_ref, da_ref, hi_ref, di_ref, u1a_ref, ma_ref,
              mi_ref, cab_ref, u2_ref, bu2_ref, o_ref):
    u1 = jnp.dot(x_ref[...], u1a_ref[...], preferred_element_type=jnp.float32)
    u1 += jnp.dot(ha_ref[0], ma_ref[0], preferred_element_type=jnp.float32)
    u1 += jnp.dot(ha_ref[1], ma_ref[1], preferred_element_type=jnp.float32)
    u1 += da_ref[...] * cab_ref[0:1, :]
    if has_inc:
        u1 += jnp.dot(hi_ref[0], mi_ref[0], preferred_element_type=jnp.float32)
        u1 += jnp.dot(hi_ref[1], mi_ref[1], preferred_element_type=jnp.float32)
        u1 += di_ref[...] * cab_ref[1:2, :]
    u1 += cab_ref[2:3, :]
    h = _silu(u1)
    o_ref[...] = x_ref[...] + _silu(
        jnp.dot(h, u2_ref[...], preferred_element_type=jnp.float32)
        + bu2_ref[...])


@functools.lru_cache(maxsize=None)
def _upd_call(n, has_inc):
    grid = n // _ROW
    return pl.pallas_call(
        functools.partial(_upd_body, has_inc),
        grid=(grid,),
        in_specs=[
            pl.BlockSpec((_ROW, _H), lambda i: (i, 0)),
            pl.BlockSpec((2, _ROW, 64), lambda i: (0, i, 0)),
            pl.BlockSpec((_ROW, 1), lambda i: (i, 0)),
            pl.BlockSpec((2, _ROW, 64), lambda i: (0, i, 0)),
            pl.BlockSpec((_ROW, 1), lambda i: (i, 0)),
            pl.BlockSpec((_H, _H), lambda i: (0, 0)),
            pl.BlockSpec((2, 64, _H), lambda i: (0, 0, 0)),
            pl.BlockSpec((2, 64, _H), lambda i: (0, 0, 0)),
            pl.BlockSpec((3, _H), lambda i: (0, 0)),
            pl.BlockSpec((_H, _H), lambda i: (0, 0)),
            pl.BlockSpec((1, _H), lambda i: (0, 0)),
        ],
        out_specs=pl.BlockSpec((_ROW, _H), lambda i: (i, 0)),
        out_shape=jax.ShapeDtypeStruct((n, _H), jnp.float32),
    )


def _pool_body(x_ref, b_ref, w1_ref, b1_ref, w2_ref, b2_ref, o_ref):
    h = _silu(jnp.dot(x_ref[...], w1_ref[...],
                      preferred_element_type=jnp.float32) + b1_ref[...])
    h = jnp.dot(h, w2_ref[...], preferred_element_type=jnp.float32) + b2_ref[...]
    gids = lax.broadcasted_iota(jnp.int32, (1, _NG), 1)
    oh = (b_ref[...] == gids).astype(jnp.float32)
    part = lax.dot_general(oh, h, (((0,), (0,)), ((), ())),
                           preferred_element_type=jnp.float32)

    @pl.when(pl.program_id(0) == 0)
    def _():
        o_ref[...] = jnp.zeros_like(o_ref)

    o_ref[...] += part


@functools.lru_cache(maxsize=None)
def _pool_call(n):
    grid = n // _ROW
    return pl.pallas_call(
        _pool_body,
        grid=(grid,),
        in_specs=[
            pl.BlockSpec((_ROW, _H), lambda i: (i, 0)),
            pl.BlockSpec((_ROW, 1), lambda i: (i, 0)),
            pl.BlockSpec((_H, _H), lambda i: (0, 0)),
            pl.BlockSpec((1, _H), lambda i: (0, 0)),
            pl.BlockSpec((_H, _H), lambda i: (0, 0)),
            pl.BlockSpec((1, _H), lambda i: (0, 0)),
        ],
        out_specs=pl.BlockSpec((_NG, _H), lambda i: (0, 0)),
        out_shape=jax.ShapeDtypeStruct((_NG, _H), jnp.float32),
    )


def _post_body(s_ref, w1_ref, b1_ref, w2_ref, b2_ref, o_ref):
    h = _silu(jnp.dot(s_ref[...], w1_ref[...],
                      preferred_element_type=jnp.float32) + b1_ref[...])
    o_ref[...] = jnp.dot(h, w2_ref[...],
                         preferred_element_type=jnp.float32) + b2_ref[...]


def _post_call():
    return pl.pallas_call(
        _post_body,
        out_shape=jax.ShapeDtypeStruct((_NG, 1), jnp.float32),
    )


def _edge_sum(ga, gb, c2, src, dst, nacc):
    out = []
    for half in range(2):
        s = _silu(ga[half][src] + gb[half][dst] + c2[half])
        out.append(jax.ops.segment_sum(s, dst, num_segments=nacc))
    return jnp.stack(out)


def _pad_to(x, m, value):
    e = x.shape[0]
    ep = -(-e // m) * m
    if ep == e:
        return x
    pad = [(0, ep - e)] + [(0, 0)] * (x.ndim - 1)
    return jnp.pad(x, pad, constant_values=value)


def kernel(x0, x1, x2, adj0, adj1, adj2, inc1, inc2, inv_adj0, inv_adj1,
           inv_adj2, inv_inc1, inv_inc2, batch0, batch1, batch2, params):
    xs = [x0, x1, x2]
    sizes = [x.shape[0] for x in xs]
    adjs = [adj0, adj1, adj2]
    incs = [inc1, inc2]
    inva = [inv_adj0, inv_adj1, inv_adj2]
    invi = [inv_inc1, inv_inc2]
    batches = [batch0, batch1, batch2]

    epad = 2048
    a_src = [_pad_to(adjs[r][0].astype(jnp.int32), epad, 0) for r in range(3)]
    a_dst = [_pad_to(adjs[r][1].astype(jnp.int32), epad, sizes[r]) for r in range(3)]
    i_src = [_pad_to(incs[i][0].astype(jnp.int32), epad, 0) for i in range(2)]
    i_dst = [_pad_to(incs[i][1].astype(jnp.int32), epad, sizes[i + 1]) for i in range(2)]
    a_inv = [_pad_to(inva[r], epad, 0.0) for r in range(3)]
    i_inv = [_pad_to(invi[i], epad, 0.0) for i in range(2)]

    deg_a = [jnp.zeros((sizes[r], 1), jnp.float32).at[adjs[r][1]].add(1.0)
             for r in range(3)]
    deg_i = [jnp.zeros((sizes[i + 1], 1), jnp.float32).at[incs[i][1]].add(1.0)
             for i in range(2)]

    xs = [_emb_call(sizes[r])(xs[r], params['emb']['w'],
                              params['emb']['b'][None, :]) for r in range(3)]

    for lp in params['layers']:
        aw = [lp['adj'][r]['l1']['w'] for r in range(3)]
        iw = [lp['inc'][i]['l1']['w'] for i in range(2)]
        w0 = jnp.concatenate([aw[0][:_H], aw[0][_H:2*_H], iw[0][:_H]], axis=1)
        w1 = jnp.concatenate([aw[1][:_H], aw[1][_H:2*_H], iw[0][_H:2*_H],
                              iw[1][:_H]], axis=1)
        w2 = jnp.concatenate([aw[2][:_H], aw[2][_H:2*_H], iw[1][_H:2*_H]],
                             axis=1)
        a0A, a0B, i1S = _proj_call(sizes[0], 3)(xs[0], w0)
        a1A, a1B, i1D, i2S = _proj_call(sizes[1], 4)(xs[1], w1)
        a2A, a2B, i2D = _proj_call(sizes[2], 3)(xs[2], w2)

        c_a = [_cprep_call(a_inv[r].shape[0])(
                   a_inv[r], aw[r][2*_H:], lp['adj'][r]['l1']['b'][None, :])
               for r in range(3)]
        c_i = [_cprep_call(i_inv[i].shape[0])(
                   i_inv[i], iw[i][2*_H:], lp['inc'][i]['l1']['b'][None, :])
               for i in range(2)]

        ha = [_edge_sum(pa, pb, c_a[r], a_src[r], a_dst[r], sizes[r])
              for r, (pa, pb) in enumerate([(a0A, a0B), (a1A, a1B), (a2A, a2B)])]
        hi = [_edge_sum(ps, pd, c_i[i], i_src[i], i_dst[i], sizes[i + 1])
              for i, (ps, pd) in enumerate([(i1S, i1D), (i2S, i2D)])]

        new_xs = []
        for r in range(3):
            u = lp['upd'][r]
            U1 = u['l1']['w']
            ma = jnp.stack([lp['adj'][r]['l2']['w'][:64] @ U1[_H:2*_H],
                            lp['adj'][r]['l2']['w'][64:] @ U1[_H:2*_H]])
            ca = lp['adj'][r]['l2']['b'] @ U1[_H:2*_H]
            if r > 0:
                mi = jnp.stack([lp['inc'][r-1]['l2']['w'][:64] @ U1[2*_H:],
                                lp['inc'][r-1]['l2']['w'][64:] @ U1[2*_H:]])
                ci = lp['inc'][r-1]['l2']['b'] @ U1[2*_H:]
                hi_in, di_in = hi[r - 1], deg_i[r - 1]
            else:
                mi = jnp.zeros((2, 64, _H), jnp.float32)
                ci = jnp.zeros((_H,), jnp.float32)
                hi_in = jnp.zeros((2, sizes[r], 64), jnp.float32)
                di_in = jnp.zeros((sizes[r], 1), jnp.float32)
            cab = jnp.stack([ca, ci, u['l1']['b']])
            new_xs.append(_upd_call(sizes[r], r > 0)(
                xs[r], ha[r], deg_a[r], hi_in, di_in, U1[:_H], ma, mi, cab,
                u['l2']['w'], u['l2']['b'][None, :]))
        xs = new_xs

    pooled = []
    for r in range(3):
        p = params['pre'][r]
        pooled.append(_pool_call(sizes[r])(
            xs[r], batches[r][:, None].astype(jnp.int32),
            p['l1']['w'], p['l1']['b'][None, :],
            p['l2']['w'], p['l2']['b'][None, :]))
    state = jnp.concatenate(pooled, axis=1)
    p = params['post']
    out = _post_call()(state, p['l1']['w'], p['l1']['b'][None, :],
                       p['l2']['w'], p['l2']['b'][None, :])
    return out[:, 0]


# speedup vs baseline: 3.5966x; 3.5966x over previous
import functools

import jax
import jax.numpy as jnp
from jax import lax
from jax.experimental import pallas as pl
from jax.experimental.pallas import tpu as pltpu
from jax.experimental.pallas import tpu_sc as plsc

_H = 128
_NG = 16
_ROW = 2000
_EROW = 2048


def _silu(x):
    return x / (1.0 + jnp.exp(-x))


def _emb_body(x_ref, w_ref, b_ref, o_ref):
    o_ref[...] = jnp.dot(x_ref[...], w_ref[...],
                         preferred_element_type=jnp.float32) + b_ref[...]


@functools.lru_cache(maxsize=None)
def _emb_call(n):
    grid = n // _ROW
    return pl.pallas_call(
        _emb_body,
        grid=(grid,),
        in_specs=[
            pl.BlockSpec((_ROW, _H), lambda i: (i, 0)),
            pl.BlockSpec((_H, _H), lambda i: (0, 0)),
            pl.BlockSpec((1, _H), lambda i: (0, 0)),
        ],
        out_specs=pl.BlockSpec((_ROW, _H), lambda i: (i, 0)),
        out_shape=jax.ShapeDtypeStruct((n, _H), jnp.float32),
    )


def _proj_body(nproj, x_ref, w_ref, *o_refs):
    y = jnp.dot(x_ref[...], w_ref[...], preferred_element_type=jnp.float32)
    for p in range(nproj):
        o_refs[p][0] = y[:, p * _H:p * _H + 64]
        o_refs[p][1] = y[:, p * _H + 64:(p + 1) * _H]


@functools.lru_cache(maxsize=None)
def _proj_call(n, nproj):
    grid = n // _ROW
    return pl.pallas_call(
        functools.partial(_proj_body, nproj),
        grid=(grid,),
        in_specs=[
            pl.BlockSpec((_ROW, _H), lambda i: (i, 0)),
            pl.BlockSpec((_H, nproj * _H), lambda i: (0, 0)),
        ],
        out_specs=[pl.BlockSpec((2, _ROW, 64), lambda i: (0, i, 0))
                   for _ in range(nproj)],
        out_shape=[jax.ShapeDtypeStruct((2, n, 64), jnp.float32)
                   for _ in range(nproj)],
    )


def _cprep_body(inv_ref, w_ref, b_ref, o_ref):
    y = jnp.dot(inv_ref[...], w_ref[...],
                preferred_element_type=jnp.float32) + b_ref[...]
    o_ref[0] = y[:, :64]
    o_ref[1] = y[:, 64:]


@functools.lru_cache(maxsize=None)
def _cprep_call(e):
    grid = e // _EROW
    return pl.pallas_call(
        _cprep_body,
        grid=(grid,),
        in_specs=[
            pl.BlockSpec((_EROW, 3), lambda i: (i, 0)),
            pl.BlockSpec((3, _H), lambda i: (0, 0)),
            pl.BlockSpec((1, _H), lambda i: (0, 0)),
        ],
        out_specs=pl.BlockSpec((2, _EROW, 64), lambda i: (0, i, 0)),
        out_shape=jax.ShapeDtypeStruct((2, e, 64), jnp.float32),
    )


def _upd_body(has_inc, x_ref, ha_ref, da_ref, hi_ref, di_ref, u1a_ref, ma_ref,
              mi_ref, cab_ref, u2_ref, bu2_ref, o_ref):
    u1 = jnp.dot(x_ref[...], u1a_ref[...], preferred_element_type=jnp.float32)
    u1 += jnp.dot(ha_ref[0], ma_ref[0], preferred_element_type=jnp.float32)
    u1 += jnp.dot(ha_ref[1], ma_ref[1], preferred_element_type=jnp.float32)
    u1 += da_ref[...] * cab_ref[0:1, :]
    if has_inc:
        u1 += jnp.dot(hi_ref[0], mi_ref[0], preferred_element_type=jnp.float32)
        u1 += jnp.dot(hi_ref[1], mi_ref[1], preferred_element_type=jnp.float32)
        u1 += di_ref[...] * cab_ref[1:2, :]
    u1 += cab_ref[2:3, :]
    h = _silu(u1)
    o_ref[...] = x_ref[...] + _silu(
        jnp.dot(h, u2_ref[...], preferred_element_type=jnp.float32)
        + bu2_ref[...])


@functools.lru_cache(maxsize=None)
def _upd_call(n, has_inc):
    grid = n // _ROW
    return pl.pallas_call(
        functools.partial(_upd_body, has_inc),
        grid=(grid,),
        in_specs=[
            pl.BlockSpec((_ROW, _H), lambda i: (i, 0)),
            pl.BlockSpec((2, _ROW, 64), lambda i: (0, i, 0)),
            pl.BlockSpec((_ROW, 1), lambda i: (i, 0)),
            pl.BlockSpec((2, _ROW, 64), lambda i: (0, i, 0)),
            pl.BlockSpec((_ROW, 1), lambda i: (i, 0)),
            pl.BlockSpec((_H, _H), lambda i: (0, 0)),
            pl.BlockSpec((2, 64, _H), lambda i: (0, 0, 0)),
            pl.BlockSpec((2, 64, _H), lambda i: (0, 0, 0)),
            pl.BlockSpec((3, _H), lambda i: (0, 0)),
            pl.BlockSpec((_H, _H), lambda i: (0, 0)),
            pl.BlockSpec((1, _H), lambda i: (0, 0)),
        ],
        out_specs=pl.BlockSpec((_ROW, _H), lambda i: (i, 0)),
        out_shape=jax.ShapeDtypeStruct((n, _H), jnp.float32),
    )


def _pool_body(x_ref, b_ref, w1_ref, b1_ref, w2_ref, b2_ref, o_ref):
    h = _silu(jnp.dot(x_ref[...], w1_ref[...],
                      preferred_element_type=jnp.float32) + b1_ref[...])
    h = jnp.dot(h, w2_ref[...], preferred_element_type=jnp.float32) + b2_ref[...]
    gids = lax.broadcasted_iota(jnp.int32, (1, _NG), 1)
    oh = (b_ref[...] == gids).astype(jnp.float32)
    part = lax.dot_general(oh, h, (((0,), (0,)), ((), ())),
                           preferred_element_type=jnp.float32)

    @pl.when(pl.program_id(0) == 0)
    def _():
        o_ref[...] = jnp.zeros_like(o_ref)

    o_ref[...] += part


@functools.lru_cache(maxsize=None)
def _pool_call(n):
    grid = n // _ROW
    return pl.pallas_call(
        _pool_body,
        grid=(grid,),
        in_specs=[
            pl.BlockSpec((_ROW, _H), lambda i: (i, 0)),
            pl.BlockSpec((_ROW, 1), lambda i: (i, 0)),
            pl.BlockSpec((_H, _H), lambda i: (0, 0)),
            pl.BlockSpec((1, _H), lambda i: (0, 0)),
            pl.BlockSpec((_H, _H), lambda i: (0, 0)),
            pl.BlockSpec((1, _H), lambda i: (0, 0)),
        ],
        out_specs=pl.BlockSpec((_NG, _H), lambda i: (0, 0)),
        out_shape=jax.ShapeDtypeStruct((_NG, _H), jnp.float32),
    )


def _post_body(s_ref, w1_ref, b1_ref, w2_ref, b2_ref, o_ref):
    h = _silu(jnp.dot(s_ref[...], w1_ref[...],
                      preferred_element_type=jnp.float32) + b1_ref[...])
    o_ref[...] = jnp.dot(h, w2_ref[...],
                         preferred_element_type=jnp.float32) + b2_ref[...]


def _post_call():
    return pl.pallas_call(
        _post_body,
        out_shape=jax.ShapeDtypeStruct((_NG, 1), jnp.float32),
    )


_NS = 16
_EB = 128


@functools.lru_cache(maxsize=None)
def _edge_sc_call(epad, nacc):
    racc = -(-(nacc + 1) // 2048) * 2048
    nblk = epad // (_NS * _EB)
    rows_tile = racc // _NS
    wb = racc // _NS
    mesh = plsc.VectorSubcoreMesh(core_axis_name="c", subcore_axis_name="s")

    @functools.partial(
        pl.kernel, mesh=mesh,
        compiler_params=pltpu.CompilerParams(use_tc_tiling_on_sc=False),
        out_type=jax.ShapeDtypeStruct((2 * racc, 64), jnp.float32),
        scratch_types=[
            pltpu.VMEM_SHARED((racc, 64), jnp.float32),
            pltpu.VMEM((_EB,), jnp.int32),
            pltpu.VMEM((_EB,), jnp.int32),
            pltpu.VMEM((_EB,), jnp.int32),
            pltpu.VMEM((_EB, 64), jnp.float32),
            pltpu.VMEM((_EB, 64), jnp.float32),
            pltpu.VMEM((_EB, 64), jnp.float32),
            pltpu.SemaphoreType.DMA,
        ])
    def k(ga, gb, c2, sidx2, didx2, dsts, out, acc, si, di, dsc, ab, bb, cb,
          sem):
        cid = lax.axis_index("c")
        sid = lax.axis_index("s")

        def zrow(r, _):
            for j in range(4):
                ab[r, pl.ds(16 * j, 16)] = jnp.zeros((16,), jnp.float32)
            return 0
        lax.fori_loop(0, _EB, zrow, 0)
        r0 = sid * rows_tile
        for q in range(rows_tile // _EB):
            pltpu.sync_copy(ab, acc.at[pl.ds(r0 + q * _EB, _EB)])
        plsc.subcore_barrier()

        ebase = sid * (nblk * _EB)

        def blk(kk, _):
            base = ebase + kk * _EB
            gbase = cid * epad + base
            pltpu.sync_copy(sidx2.at[pl.ds(gbase, _EB)], si)
            pltpu.sync_copy(didx2.at[pl.ds(gbase, _EB)], di)
            pltpu.sync_copy(dsts.at[pl.ds(base, _EB)], dsc)
            ca = pltpu.async_copy(ga.at[si], ab, sem)
            cbc = pltpu.async_copy(gb.at[di], bb, sem)
            pltpu.sync_copy(c2.at[pl.ds(gbase, _EB)], cb)
            ca.wait()
            cbc.wait()

            def crow(r, _):
                for j in range(4):
                    sl = pl.ds(16 * j, 16)
                    s = ab[r, sl] + bb[r, sl] + cb[r, sl]
                    ab[r, sl] = s / (1.0 + jnp.exp(-s))
                return 0
            lax.fori_loop(0, _EB, crow, 0)
            pltpu.sync_copy(ab, acc.at[dsc], add=True)
            return 0
        lax.fori_loop(0, nblk, blk, 0)
        plsc.subcore_barrier()

        pltpu.sync_copy(acc.at[pl.ds(sid * wb, wb)],
                        out.at[pl.ds(cid * racc + sid * wb, wb)])

    return k


def _edge_sum(ga, gb, c2, sidx2, didx2, dsts, nacc):
    epad = dsts.shape[0]
    racc = -(-(nacc + 1) // 2048) * 2048
    gaf = ga.reshape(-1, 64)
    gbf = gb.reshape(-1, 64)
    c2f = c2.reshape(-1, 64)
    h = _edge_sc_call(epad, nacc)(gaf, gbf, c2f, sidx2, didx2, dsts)
    return h.reshape(2, racc, 64)[:, :nacc]


def _pad_to(x, m, value):
    e = x.shape[0]
    ep = -(-e // m) * m
    if ep == e:
        return x
    pad = [(0, ep - e)] + [(0, 0)] * (x.ndim - 1)
    return jnp.pad(x, pad, constant_values=value)


def kernel(x0, x1, x2, adj0, adj1, adj2, inc1, inc2, inv_adj0, inv_adj1,
           inv_adj2, inv_inc1, inv_inc2, batch0, batch1, batch2, params):
    xs = [x0, x1, x2]
    sizes = [x.shape[0] for x in xs]
    adjs = [adj0, adj1, adj2]
    incs = [inc1, inc2]
    inva = [inv_adj0, inv_adj1, inv_adj2]
    invi = [inv_inc1, inv_inc2]
    batches = [batch0, batch1, batch2]

    epad = 2048
    a_src = [_pad_to(adjs[r][0].astype(jnp.int32), epad, 0) for r in range(3)]
    a_dst = [_pad_to(adjs[r][1].astype(jnp.int32), epad, sizes[r]) for r in range(3)]
    i_src = [_pad_to(incs[i][0].astype(jnp.int32), epad, 0) for i in range(2)]
    i_dst = [_pad_to(incs[i][1].astype(jnp.int32), epad, sizes[i + 1]) for i in range(2)]
    a_inv = [_pad_to(inva[r], epad, 0.0) for r in range(3)]
    i_inv = [_pad_to(invi[i], epad, 0.0) for i in range(2)]

    deg_a = [jnp.zeros((sizes[r], 1), jnp.float32).at[adjs[r][1]].add(1.0)
             for r in range(3)]
    deg_i = [jnp.zeros((sizes[i + 1], 1), jnp.float32).at[incs[i][1]].add(1.0)
             for i in range(2)]

    a_dstg = [_pad_to(adjs[r][1].astype(jnp.int32), epad, 0) for r in range(3)]
    i_dstg = [_pad_to(incs[i][1].astype(jnp.int32), epad, 0) for i in range(2)]
    a_sg = [jnp.concatenate([a_src[r], a_src[r] + sizes[r]]) for r in range(3)]
    a_dg = [jnp.concatenate([a_dstg[r], a_dstg[r] + sizes[r]])
            for r in range(3)]
    i_sg = [jnp.concatenate([i_src[i], i_src[i] + sizes[i]]) for i in range(2)]
    i_dg = [jnp.concatenate([i_dstg[i], i_dstg[i] + sizes[i + 1]])
            for i in range(2)]

    xs = [_emb_call(sizes[r])(xs[r], params['emb']['w'],
                              params['emb']['b'][None, :]) for r in range(3)]

    for lp in params['layers']:
        aw = [lp['adj'][r]['l1']['w'] for r in range(3)]
        iw = [lp['inc'][i]['l1']['w'] for i in range(2)]
        w0 = jnp.concatenate([aw[0][:_H], aw[0][_H:2*_H], iw[0][:_H]], axis=1)
        w1 = jnp.concatenate([aw[1][:_H], aw[1][_H:2*_H], iw[0][_H:2*_H],
                              iw[1][:_H]], axis=1)
        w2 = jnp.concatenate([aw[2][:_H], aw[2][_H:2*_H], iw[1][_H:2*_H]],
                             axis=1)
        a0A, a0B, i1S = _proj_call(sizes[0], 3)(xs[0], w0)
        a1A, a1B, i1D, i2S = _proj_call(sizes[1], 4)(xs[1], w1)
        a2A, a2B, i2D = _proj_call(sizes[2], 3)(xs[2], w2)

        c_a = [_cprep_call(a_inv[r].shape[0])(
                   a_inv[r], aw[r][2*_H:], lp['adj'][r]['l1']['b'][None, :])
               for r in range(3)]
        c_i = [_cprep_call(i_inv[i].shape[0])(
                   i_inv[i], iw[i][2*_H:], lp['inc'][i]['l1']['b'][None, :])
               for i in range(2)]

        ha = [_edge_sum(pa, pb, c_a[r], a_sg[r], a_dg[r], a_dst[r], sizes[r])
              for r, (pa, pb) in enumerate([(a0A, a0B), (a1A, a1B), (a2A, a2B)])]
        hi = [_edge_sum(ps, pd, c_i[i], i_sg[i], i_dg[i], i_dst[i],
                        sizes[i + 1])
              for i, (ps, pd) in enumerate([(i1S, i1D), (i2S, i2D)])]

        new_xs = []
        for r in range(3):
            u = lp['upd'][r]
            U1 = u['l1']['w']
            ma = jnp.stack([lp['adj'][r]['l2']['w'][:64] @ U1[_H:2*_H],
                            lp['adj'][r]['l2']['w'][64:] @ U1[_H:2*_H]])
            ca = lp['adj'][r]['l2']['b'] @ U1[_H:2*_H]
            if r > 0:
                mi = jnp.stack([lp['inc'][r-1]['l2']['w'][:64] @ U1[2*_H:],
                                lp['inc'][r-1]['l2']['w'][64:] @ U1[2*_H:]])
                ci = lp['inc'][r-1]['l2']['b'] @ U1[2*_H:]
                hi_in, di_in = hi[r - 1], deg_i[r - 1]
            else:
                mi = jnp.zeros((2, 64, _H), jnp.float32)
                ci = jnp.zeros((_H,), jnp.float32)
                hi_in = jnp.zeros((2, sizes[r], 64), jnp.float32)
                di_in = jnp.zeros((sizes[r], 1), jnp.float32)
            cab = jnp.stack([ca, ci, u['l1']['b']])
            new_xs.append(_upd_call(sizes[r], r > 0)(
                xs[r], ha[r], deg_a[r], hi_in, di_in, U1[:_H], ma, mi, cab,
                u['l2']['w'], u['l2']['b'][None, :]))
        xs = new_xs

    pooled = []
    for r in range(3):
        p = params['pre'][r]
        pooled.append(_pool_call(sizes[r])(
            xs[r], batches[r][:, None].astype(jnp.int32),
            p['l1']['w'], p['l1']['b'][None, :],
            p['l2']['w'], p['l2']['b'][None, :]))
    state = jnp.concatenate(pooled, axis=1)
    p = params['post']
    out = _post_call()(state, p['l1']['w'], p['l1']['b'][None, :],
                       p['l2']['w'], p['l2']['b'][None, :])
    return out[:, 0]


# speedup vs baseline: 4.1749x; 1.1608x over previous
import functools

import jax
import jax.numpy as jnp
from jax import lax
from jax.experimental import pallas as pl
from jax.experimental.pallas import tpu as pltpu
from jax.experimental.pallas import tpu_sc as plsc

_H = 128
_NG = 16
_ROW = 2000
_EROW = 2048


def _silu(x):
    return x / (1.0 + jnp.exp(-x))


def _emb_body(x_ref, w_ref, b_ref, o_ref):
    o_ref[...] = jnp.dot(x_ref[...], w_ref[...],
                         preferred_element_type=jnp.float32) + b_ref[...]


@functools.lru_cache(maxsize=None)
def _emb_call(n):
    grid = n // _ROW
    return pl.pallas_call(
        _emb_body,
        grid=(grid,),
        in_specs=[
            pl.BlockSpec((_ROW, _H), lambda i: (i, 0)),
            pl.BlockSpec((_H, _H), lambda i: (0, 0)),
            pl.BlockSpec((1, _H), lambda i: (0, 0)),
        ],
        out_specs=pl.BlockSpec((_ROW, _H), lambda i: (i, 0)),
        out_shape=jax.ShapeDtypeStruct((n, _H), jnp.float32),
    )


def _proj_body(nproj, x_ref, w_ref, *o_refs):
    y = jnp.dot(x_ref[...], w_ref[...], preferred_element_type=jnp.float32)
    for p in range(nproj):
        o_refs[p][0] = y[:, p * _H:p * _H + 64]
        o_refs[p][1] = y[:, p * _H + 64:(p + 1) * _H]


@functools.lru_cache(maxsize=None)
def _proj_call(n, nproj):
    grid = n // _ROW
    return pl.pallas_call(
        functools.partial(_proj_body, nproj),
        grid=(grid,),
        in_specs=[
            pl.BlockSpec((_ROW, _H), lambda i: (i, 0)),
            pl.BlockSpec((_H, nproj * _H), lambda i: (0, 0)),
        ],
        out_specs=[pl.BlockSpec((2, _ROW, 64), lambda i: (0, i, 0))
                   for _ in range(nproj)],
        out_shape=[jax.ShapeDtypeStruct((2, n, 64), jnp.float32)
                   for _ in range(nproj)],
    )


def _cprep_body(inv_ref, w_ref, b_ref, o_ref):
    y = jnp.dot(inv_ref[...], w_ref[...],
                preferred_element_type=jnp.float32) + b_ref[...]
    o_ref[0] = y[:, :64]
    o_ref[1] = y[:, 64:]


@functools.lru_cache(maxsize=None)
def _cprep_call(e):
    grid = e // _EROW
    return pl.pallas_call(
        _cprep_body,
        grid=(grid,),
        in_specs=[
            pl.BlockSpec((_EROW, 3), lambda i: (i, 0)),
            pl.BlockSpec((3, _H), lambda i: (0, 0)),
            pl.BlockSpec((1, _H), lambda i: (0, 0)),
        ],
        out_specs=pl.BlockSpec((2, _EROW, 64), lambda i: (0, i, 0)),
        out_shape=jax.ShapeDtypeStruct((2, e, 64), jnp.float32),
    )


def _upd_body(has_inc, x_ref, ha_ref, da_ref, hi_ref, di_ref, u1a_ref, ma_ref,
              mi_ref, cab_ref, u2_ref, bu2_ref, o_ref):
    u1 = jnp.dot(x_ref[...], u1a_ref[...], preferred_element_type=jnp.float32)
    u1 += jnp.dot(ha_ref[0], ma_ref[0], preferred_element_type=jnp.float32)
    u1 += jnp.dot(ha_ref[1], ma_ref[1], preferred_element_type=jnp.float32)
    u1 += da_ref[...] * cab_ref[0:1, :]
    if has_inc:
        u1 += jnp.dot(hi_ref[0], mi_ref[0], preferred_element_type=jnp.float32)
        u1 += jnp.dot(hi_ref[1], mi_ref[1], preferred_element_type=jnp.float32)
        u1 += di_ref[...] * cab_ref[1:2, :]
    u1 += cab_ref[2:3, :]
    h = _silu(u1)
    o_ref[...] = x_ref[...] + _silu(
        jnp.dot(h, u2_ref[...], preferred_element_type=jnp.float32)
        + bu2_ref[...])


@functools.lru_cache(maxsize=None)
def _upd_call(n, has_inc):
    grid = n // _ROW
    return pl.pallas_call(
        functools.partial(_upd_body, has_inc),
        grid=(grid,),
        in_specs=[
            pl.BlockSpec((_ROW, _H), lambda i: (i, 0)),
            pl.BlockSpec((2, _ROW, 64), lambda i: (0, i, 0)),
            pl.BlockSpec((_ROW, 1), lambda i: (i, 0)),
            pl.BlockSpec((2, _ROW, 64), lambda i: (0, i, 0)),
            pl.BlockSpec((_ROW, 1), lambda i: (i, 0)),
            pl.BlockSpec((_H, _H), lambda i: (0, 0)),
            pl.BlockSpec((2, 64, _H), lambda i: (0, 0, 0)),
            pl.BlockSpec((2, 64, _H), lambda i: (0, 0, 0)),
            pl.BlockSpec((3, _H), lambda i: (0, 0)),
            pl.BlockSpec((_H, _H), lambda i: (0, 0)),
            pl.BlockSpec((1, _H), lambda i: (0, 0)),
        ],
        out_specs=pl.BlockSpec((_ROW, _H), lambda i: (i, 0)),
        out_shape=jax.ShapeDtypeStruct((n, _H), jnp.float32),
    )


def _pool_body(x_ref, b_ref, w1_ref, b1_ref, w2_ref, b2_ref, o_ref):
    h = _silu(jnp.dot(x_ref[...], w1_ref[...],
                      preferred_element_type=jnp.float32) + b1_ref[...])
    h = jnp.dot(h, w2_ref[...], preferred_element_type=jnp.float32) + b2_ref[...]
    gids = lax.broadcasted_iota(jnp.int32, (1, _NG), 1)
    oh = (b_ref[...] == gids).astype(jnp.float32)
    part = lax.dot_general(oh, h, (((0,), (0,)), ((), ())),
                           preferred_element_type=jnp.float32)

    @pl.when(pl.program_id(0) == 0)
    def _():
        o_ref[...] = jnp.zeros_like(o_ref)

    o_ref[...] += part


@functools.lru_cache(maxsize=None)
def _pool_call(n):
    grid = n // _ROW
    return pl.pallas_call(
        _pool_body,
        grid=(grid,),
        in_specs=[
            pl.BlockSpec((_ROW, _H), lambda i: (i, 0)),
            pl.BlockSpec((_ROW, 1), lambda i: (i, 0)),
            pl.BlockSpec((_H, _H), lambda i: (0, 0)),
            pl.BlockSpec((1, _H), lambda i: (0, 0)),
            pl.BlockSpec((_H, _H), lambda i: (0, 0)),
            pl.BlockSpec((1, _H), lambda i: (0, 0)),
        ],
        out_specs=pl.BlockSpec((_NG, _H), lambda i: (0, 0)),
        out_shape=jax.ShapeDtypeStruct((_NG, _H), jnp.float32),
    )


def _post_body(s_ref, w1_ref, b1_ref, w2_ref, b2_ref, o_ref):
    h = _silu(jnp.dot(s_ref[...], w1_ref[...],
                      preferred_element_type=jnp.float32) + b1_ref[...])
    o_ref[...] = jnp.dot(h, w2_ref[...],
                         preferred_element_type=jnp.float32) + b2_ref[...]


def _post_call():
    return pl.pallas_call(
        _post_body,
        out_shape=jax.ShapeDtypeStruct((_NG, 1), jnp.float32),
    )


_NS = 16
_EB = 128


def _eb_for(nacc):
    return 128 if nacc <= 16000 else 64


@functools.lru_cache(maxsize=None)
def _edge_sc_call(epad, nacc):
    eb = _eb_for(nacc)
    racc = -(-(nacc + 1) // 2048) * 2048
    nblk = epad // (_NS * eb)
    assert nblk % 2 == 0 and nblk >= 4
    nblk_tot = epad // eb
    rows_tile = racc // _NS
    mesh = plsc.VectorSubcoreMesh(core_axis_name="c", subcore_axis_name="s")

    @functools.partial(
        pl.kernel, mesh=mesh,
        compiler_params=pltpu.CompilerParams(use_tc_tiling_on_sc=False),
        out_type=jax.ShapeDtypeStruct((2 * racc, 64), jnp.float32),
        scratch_types=[
            pltpu.VMEM_SHARED((racc, 64), jnp.float32),
            [pltpu.VMEM((3, eb), jnp.int32) for _ in range(2)],
            [pltpu.VMEM((eb, 64), jnp.float32) for _ in range(2)],
            [pltpu.VMEM((eb, 64), jnp.float32) for _ in range(2)],
            [pltpu.VMEM((eb, 64), jnp.float32) for _ in range(2)],
            [pltpu.VMEM((eb, 64), jnp.float32) for _ in range(2)],
            [pltpu.VMEM((eb,), jnp.int32) for _ in range(2)],
            [pltpu.SemaphoreType.DMA for _ in range(2)],
            [pltpu.SemaphoreType.DMA for _ in range(2)],
        ])
    def k(ga, gb, c2, idxall, out, acc, ib, ab, bb, cb, ob, dsc, semg, sems):
        cid = lax.axis_index("c")
        sid = lax.axis_index("s")

        def zrow(r, _):
            for j in range(4):
                ab[0][r, pl.ds(16 * j, 16)] = jnp.zeros((16,), jnp.float32)
            return 0
        lax.fori_loop(0, eb, zrow, 0)
        r0 = sid * rows_tile
        for q in range(rows_tile // eb):
            pltpu.sync_copy(ab[0], acc.at[pl.ds(r0 + q * eb, eb)])
        plsc.subcore_barrier()

        blk0 = sid * nblk

        def issue(kk, s):
            gblk = cid * nblk_tot + blk0 + kk
            pltpu.sync_copy(idxall.at[gblk], ib[s])
            pltpu.async_copy(ga.at[ib[s].at[0]], ab[s], semg[s])
            pltpu.async_copy(gb.at[ib[s].at[1]], bb[s], semg[s])
            pltpu.async_copy(c2.at[pl.ds((gblk) * eb, eb)], cb[s], semg[s])

        def finish(kk, s, first):
            pltpu.make_async_copy(ga.at[ib[s].at[0]], ab[s], semg[s]).wait()
            pltpu.make_async_copy(gb.at[ib[s].at[1]], bb[s], semg[s]).wait()
            pltpu.make_async_copy(c2.at[pl.ds(0, eb)], cb[s], semg[s]).wait()
            if not first:
                pltpu.make_async_copy(ob[s], acc.at[dsc[s]], sems[s]).wait()

            def crow(r, _):
                for j in range(4):
                    sl = pl.ds(16 * j, 16)
                    v = ab[s][r, sl] + bb[s][r, sl] + cb[s][r, sl]
                    ob[s][r, sl] = v / (1.0 + jnp.exp(-v))
                return 0
            lax.fori_loop(0, eb, crow, 0)
            for j in range(eb // 16):
                dsc[s][pl.ds(16 * j, 16)] = ib[s][2, pl.ds(16 * j, 16)]
            pltpu.async_copy(ob[s], acc.at[dsc[s]], sems[s], add=True)

        issue(0, 0)
        issue(1, 1)

        def pair(p, _):
            kk = 2 * p

            def fin0(first):
                finish(kk, 0, first)
                issue(kk + 2, 0)
                finish(kk + 1, 1, first)
                issue(kk + 3, 1)

            @pl.when(p == 0)
            def _():
                fin0(True)

            @pl.when(p > 0)
            def _():
                fin0(False)
            return 0
        lax.fori_loop(0, nblk // 2 - 1, pair, 0)
        finish(nblk - 2, 0, False)
        finish(nblk - 1, 1, False)
        pltpu.make_async_copy(ob[0], acc.at[dsc[0]], sems[0]).wait()
        pltpu.make_async_copy(ob[1], acc.at[dsc[1]], sems[1]).wait()
        plsc.subcore_barrier()

        pltpu.sync_copy(acc.at[pl.ds(r0, rows_tile)],
                        out.at[pl.ds(cid * racc + r0, rows_tile)])

    return k


def _edge_sum(ga, gb, c2, idxall, nacc):
    racc = -(-(nacc + 1) // 2048) * 2048
    epad = idxall.shape[0] // 2 * _eb_for(nacc)
    gaf = ga.reshape(-1, 64)
    gbf = gb.reshape(-1, 64)
    c2f = c2.reshape(-1, 64)
    h = _edge_sc_call(epad, nacc)(gaf, gbf, c2f, idxall)
    return h.reshape(2, racc, 64)[:, :nacc]


def _pad_to(x, m, value):
    e = x.shape[0]
    ep = -(-e // m) * m
    if ep == e:
        return x
    pad = [(0, ep - e)] + [(0, 0)] * (x.ndim - 1)
    return jnp.pad(x, pad, constant_values=value)


def kernel(x0, x1, x2, adj0, adj1, adj2, inc1, inc2, inv_adj0, inv_adj1,
           inv_adj2, inv_inc1, inv_inc2, batch0, batch1, batch2, params):
    xs = [x0, x1, x2]
    sizes = [x.shape[0] for x in xs]
    adjs = [adj0, adj1, adj2]
    incs = [inc1, inc2]
    inva = [inv_adj0, inv_adj1, inv_adj2]
    invi = [inv_inc1, inv_inc2]
    batches = [batch0, batch1, batch2]

    epad = 4096
    a_src = [_pad_to(adjs[r][0].astype(jnp.int32), epad, 0) for r in range(3)]
    a_dst = [_pad_to(adjs[r][1].astype(jnp.int32), epad, sizes[r]) for r in range(3)]
    i_src = [_pad_to(incs[i][0].astype(jnp.int32), epad, 0) for i in range(2)]
    i_dst = [_pad_to(incs[i][1].astype(jnp.int32), epad, sizes[i + 1]) for i in range(2)]
    a_inv = [_pad_to(inva[r], epad, 0.0) for r in range(3)]
    i_inv = [_pad_to(invi[i], epad, 0.0) for i in range(2)]

    deg_a = [jnp.zeros((sizes[r], 1), jnp.float32).at[adjs[r][1]].add(1.0)
             for r in range(3)]
    deg_i = [jnp.zeros((sizes[i + 1], 1), jnp.float32).at[incs[i][1]].add(1.0)
             for i in range(2)]

    def _pack_idx(srcp, dst_raw, dst_scat, na, nb, eb):
        nb_tot = srcp.shape[0] // eb

        def plane(oa, ob_):
            return jnp.stack([(srcp + oa).reshape(nb_tot, eb),
                              (dst_raw + ob_).reshape(nb_tot, eb),
                              dst_scat.reshape(nb_tot, eb)], axis=1)
        return jnp.concatenate([plane(0, 0), plane(na, nb)], axis=0)

    a_dstg = [_pad_to(adjs[r][1].astype(jnp.int32), epad, 0) for r in range(3)]
    i_dstg = [_pad_to(incs[i][1].astype(jnp.int32), epad, 0) for i in range(2)]
    idx_a = [_pack_idx(a_src[r], a_dstg[r], a_dst[r], sizes[r], sizes[r],
                       _eb_for(sizes[r]))
             for r in range(3)]
    idx_i = [_pack_idx(i_src[i], i_dstg[i], i_dst[i], sizes[i], sizes[i + 1],
                       _eb_for(sizes[i + 1]))
             for i in range(2)]

    xs = [_emb_call(sizes[r])(xs[r], params['emb']['w'],
                              params['emb']['b'][None, :]) for r in range(3)]

    for lp in params['layers']:
        aw = [lp['adj'][r]['l1']['w'] for r in range(3)]
        iw = [lp['inc'][i]['l1']['w'] for i in range(2)]
        w0 = jnp.concatenate([aw[0][:_H], aw[0][_H:2*_H], iw[0][:_H]], axis=1)
        w1 = jnp.concatenate([aw[1][:_H], aw[1][_H:2*_H], iw[0][_H:2*_H],
                              iw[1][:_H]], axis=1)
        w2 = jnp.concatenate([aw[2][:_H], aw[2][_H:2*_H], iw[1][_H:2*_H]],
                             axis=1)
        a0A, a0B, i1S = _proj_call(sizes[0], 3)(xs[0], w0)
        a1A, a1B, i1D, i2S = _proj_call(sizes[1], 4)(xs[1], w1)
        a2A, a2B, i2D = _proj_call(sizes[2], 3)(xs[2], w2)

        c_a = [_cprep_call(a_inv[r].shape[0])(
                   a_inv[r], aw[r][2*_H:], lp['adj'][r]['l1']['b'][None, :])
               for r in range(3)]
        c_i = [_cprep_call(i_inv[i].shape[0])(
                   i_inv[i], iw[i][2*_H:], lp['inc'][i]['l1']['b'][None, :])
               for i in range(2)]

        ha = [_edge_sum(pa, pb, c_a[r], idx_a[r], sizes[r])
              for r, (pa, pb) in enumerate([(a0A, a0B), (a1A, a1B), (a2A, a2B)])]
        hi = [_edge_sum(ps, pd, c_i[i], idx_i[i], sizes[i + 1])
              for i, (ps, pd) in enumerate([(i1S, i1D), (i2S, i2D)])]

        new_xs = []
        for r in range(3):
            u = lp['upd'][r]
            U1 = u['l1']['w']
            ma = jnp.stack([lp['adj'][r]['l2']['w'][:64] @ U1[_H:2*_H],
                            lp['adj'][r]['l2']['w'][64:] @ U1[_H:2*_H]])
            ca = lp['adj'][r]['l2']['b'] @ U1[_H:2*_H]
            if r > 0:
                mi = jnp.stack([lp['inc'][r-1]['l2']['w'][:64] @ U1[2*_H:],
                                lp['inc'][r-1]['l2']['w'][64:] @ U1[2*_H:]])
                ci = lp['inc'][r-1]['l2']['b'] @ U1[2*_H:]
                hi_in, di_in = hi[r - 1], deg_i[r - 1]
            else:
                mi = jnp.zeros((2, 64, _H), jnp.float32)
                ci = jnp.zeros((_H,), jnp.float32)
                hi_in = jnp.zeros((2, sizes[r], 64), jnp.float32)
                di_in = jnp.zeros((sizes[r], 1), jnp.float32)
            cab = jnp.stack([ca, ci, u['l1']['b']])
            new_xs.append(_upd_call(sizes[r], r > 0)(
                xs[r], ha[r], deg_a[r], hi_in, di_in, U1[:_H], ma, mi, cab,
                u['l2']['w'], u['l2']['b'][None, :]))
        xs = new_xs

    pooled = []
    for r in range(3):
        p = params['pre'][r]
        pooled.append(_pool_call(sizes[r])(
            xs[r], batches[r][:, None].astype(jnp.int32),
            p['l1']['w'], p['l1']['b'][None, :],
            p['l2']['w'], p['l2']['b'][None, :]))
    state = jnp.concatenate(pooled, axis=1)
    p = params['post']
    out = _post_call()(state, p['l1']['w'], p['l1']['b'][None, :],
                       p['l2']['w'], p['l2']['b'][None, :])
    return out[:, 0]
